# Initial kernel scaffold; baseline (speedup 1.0000x reference)
import jax
import jax.numpy as jnp
from jax.experimental import pallas as pl


def kernel(feat_0, feat_1, feat_2, feat_3, feat_4, feat_5, feat_6, W_tl_heat, W_br_heat, W_tl_regr, W_br_regr):
    raise NotImplementedError("write your pallas kernel here")


# speedup vs baseline: 1.0248x; 1.0248x over previous
import functools
import jax
import jax.numpy as jnp
from jax import lax
from jax.experimental import pallas as pl

_K = 100
_NCLS = 80
_CF = 256
_SIZES = [(64, 64), (64, 32), (32, 64), (32, 32), (32, 16), (16, 32), (16, 16)]
_NL = len(_SIZES)
_PH = 64
_PW = 64
_PP = _PH * _PW
_NEG = -1e9


def _head_nms_body(x_ref, w_ref, o_ref):
    l = pl.program_id(0)
    x = x_ref[0]
    w = w_ref[...]
    logits = jnp.dot(x, w, preferred_element_type=jnp.float32)

    heat = jnp.clip(jax.nn.sigmoid(logits[:, :2 * _NCLS]), 1e-4, 1.0 - 1e-4)

    h_l = sum((l == i) * h for i, (h, w_) in enumerate(_SIZES))
    w_l = sum((l == i) * w_ for i, (h, w_) in enumerate(_SIZES))

    r = lax.broadcasted_iota(jnp.int32, (_PP, 1), 0)
    yy = r // _PW
    xx = r % _PW
    valid = (yy < h_l) & (xx < w_l)

    hv = jnp.where(valid, heat, _NEG)

    neg_row = jnp.full((1, 2 * _NCLS), _NEG, jnp.float32)
    left = jnp.where(xx == 0, _NEG,
                     jnp.concatenate([neg_row, hv[:-1, :]], axis=0))
    right = jnp.where(xx == _PW - 1, _NEG,
                      jnp.concatenate([hv[1:, :], neg_row], axis=0))
    hx = jnp.maximum(hv, jnp.maximum(left, right))
    neg_blk = jnp.full((_PW, 2 * _NCLS), _NEG, jnp.float32)
    up = jnp.concatenate([neg_blk, hx[:-_PW, :]], axis=0)
    dn = jnp.concatenate([hx[_PW:, :], neg_blk], axis=0)
    hmax = jnp.maximum(hx, jnp.maximum(up, dn))

    keep = (hmax == hv).astype(jnp.float32)
    scores = jnp.where(valid, heat * keep, 0.0)

    o_ref[0] = jnp.concatenate([scores, logits[:, 2 * _NCLS:]], axis=1)


@jax.jit
def _heads(feats, w_all):
    return pl.pallas_call(
        _head_nms_body,
        grid=(_NL,),
        in_specs=[
            pl.BlockSpec((1, _PP, _CF), lambda l: (l, 0, 0)),
            pl.BlockSpec((_CF, _CF), lambda l: (0, 0)),
        ],
        out_specs=pl.BlockSpec((1, _PP, _CF), lambda l: (l, 0, 0)),
        out_shape=jax.ShapeDtypeStruct((_NL, _PP, _CF), jnp.float32),
    )(feats, w_all)


def kernel(feat_0, feat_1, feat_2, feat_3, feat_4, feat_5, feat_6,
           W_tl_heat, W_br_heat, W_tl_regr, W_br_regr):
    feats_in = [feat_0, feat_1, feat_2, feat_3, feat_4, feat_5, feat_6]
    padded = []
    for f, (h, w) in zip(feats_in, _SIZES):
        fm = jnp.transpose(f[0], (1, 2, 0))
        fm = jnp.pad(fm, ((0, _PH - h), (0, _PW - w), (0, 0)))
        padded.append(fm.reshape(_PP, _CF))
    feats = jnp.stack(padded, axis=0)
    w_all = jnp.concatenate(
        [W_tl_heat, W_br_heat, W_tl_regr, W_br_regr,
         jnp.zeros((_CF, _CF - 2 * _NCLS - 4), jnp.float32)], axis=1)

    proc = _heads(feats, w_all)

    all_scores = []
    all_boxes = []
    for l, (h, w) in enumerate(_SIZES):
        blk = proc[l].reshape(_PH, _PW, _CF)[:h, :w]
        tl_sc = jnp.transpose(blk[:, :, :_NCLS], (2, 0, 1)).reshape(1, -1)
        br_sc = jnp.transpose(blk[:, :, _NCLS:2 * _NCLS], (2, 0, 1)).reshape(1, -1)
        regr = blk[:, :, 2 * _NCLS:2 * _NCLS + 4].reshape(h * w, 4)

        tl_s, tl_i = lax.top_k(tl_sc, _K)
        br_s, br_i = lax.top_k(br_sc, _K)
        tl_c = (tl_i // (h * w)).astype(jnp.int32)
        br_c = (br_i // (h * w)).astype(jnp.int32)
        tl_p = tl_i % (h * w)
        br_p = br_i % (h * w)
        tl_y = (tl_p // w).astype(jnp.float32)
        tl_x = (tl_p % w).astype(jnp.float32)
        br_y = (br_p // w).astype(jnp.float32)
        br_x = (br_p % w).astype(jnp.float32)
        tl_r = regr[tl_p[0]]
        br_r = regr[br_p[0]]
        tl_xs = (tl_x[0] + tl_r[:, 0])[:, None]
        tl_ys = (tl_y[0] + tl_r[:, 1])[:, None]
        br_xs = (br_x[0] + br_r[:, 2])[None, :]
        br_ys = (br_y[0] + br_r[:, 3])[None, :]
        tl_xs = jnp.broadcast_to(tl_xs, (_K, _K))
        tl_ys = jnp.broadcast_to(tl_ys, (_K, _K))
        br_xs = jnp.broadcast_to(br_xs, (_K, _K))
        br_ys = jnp.broadcast_to(br_ys, (_K, _K))
        bboxes = jnp.stack((tl_xs, tl_ys, br_xs, br_ys), axis=2)
        scores = (tl_s[0][:, None] + br_s[0][None, :]) / 2.0
        cls_mask = (tl_c[0][:, None] != br_c[0][None, :])
        geo_mask = (br_xs < tl_xs) | (br_ys < tl_ys)
        scores = jnp.where(cls_mask | geo_mask, -1.0, scores)
        all_scores.append(scores.reshape(1, -1))
        all_boxes.append(bboxes.reshape(1, -1, 4))

    scores = jnp.concatenate(all_scores, axis=1)
    boxes = jnp.concatenate(all_boxes, axis=1)
    det_scores, det_inds = lax.top_k(scores, 1000)
    det_boxes = jnp.take_along_axis(
        boxes, det_inds[:, :, None].repeat(4, axis=2), axis=1)
    return det_scores, det_boxes


# speedup vs baseline: 1.4314x; 1.3967x over previous
import functools
import jax
import jax.numpy as jnp
from jax import lax
from jax.experimental import pallas as pl
from jax.experimental.pallas import tpu as pltpu

_K = 100
_NCLS = 80
_CF = 256
_SIZES = [(64, 64), (64, 32), (32, 64), (32, 32), (32, 16), (16, 32), (16, 16)]
_NL = len(_SIZES)
_PH = 64
_PW = 64
_PP = _PH * _PW
_NEG = -1e9
_BIG = 2 ** 30


def _head_nms_body(x_ref, w_ref, o_ref):
    l = pl.program_id(0)
    x = x_ref[0]
    w = w_ref[...]
    logits = jnp.dot(x, w, preferred_element_type=jnp.float32)

    heat = jnp.clip(jax.nn.sigmoid(logits[:, :2 * _NCLS]), 1e-4, 1.0 - 1e-4)

    h_l = sum((l == i) * h for i, (h, _) in enumerate(_SIZES))
    w_l = sum((l == i) * w_ for i, (_, w_) in enumerate(_SIZES))

    r = lax.broadcasted_iota(jnp.int32, (_PP, 1), 0)
    yy = r // _PW
    xx = r % _PW
    valid = (yy < h_l) & (xx < w_l)

    hv = jnp.where(valid, heat, _NEG)

    neg_row = jnp.full((1, 2 * _NCLS), _NEG, jnp.float32)
    left = jnp.where(xx == 0, _NEG,
                     jnp.concatenate([neg_row, hv[:-1, :]], axis=0))
    right = jnp.where(xx == _PW - 1, _NEG,
                      jnp.concatenate([hv[1:, :], neg_row], axis=0))
    hx = jnp.maximum(hv, jnp.maximum(left, right))
    neg_blk = jnp.full((_PW, 2 * _NCLS), _NEG, jnp.float32)
    up = jnp.concatenate([neg_blk, hx[:-_PW, :]], axis=0)
    dn = jnp.concatenate([hx[_PW:, :], neg_blk], axis=0)
    hmax = jnp.maximum(hx, jnp.maximum(up, dn))

    keep = (hmax == hv).astype(jnp.float32)
    scores = jnp.where(valid, heat * keep, 0.0)

    o_ref[0] = jnp.concatenate([scores, logits[:, 2 * _NCLS:]], axis=1)


def _heads(feats, w_all):
    return pl.pallas_call(
        _head_nms_body,
        grid=(_NL,),
        in_specs=[
            pl.BlockSpec((1, _PP, _CF), lambda l: (l, 0, 0)),
            pl.BlockSpec((_CF, _CF), lambda l: (0, 0)),
        ],
        out_specs=pl.BlockSpec((1, _PP, _CF), lambda l: (l, 0, 0)),
        out_shape=jax.ShapeDtypeStruct((_NL, _PP, _CF), jnp.float32),
    )(feats, w_all)


def _ext_step(k, carry, xs_ref, nb, forms):
    sub8 = lax.broadcasted_iota(jnp.int32, (8, 128), 0)
    lane8 = lax.broadcasted_iota(jnp.int32, (8, 128), 1)
    flat8 = sub8 * 128 + lane8
    biota = (lax.broadcasted_iota(jnp.int32, (nb, 128), 0) * 128 +
             lax.broadcasted_iota(jnp.int32, (nb, 128), 1))
    rowb = lax.broadcasted_iota(jnp.int32, (nb, 1), 0)

    if forms:
        m_sum, vacc, iacc, vcol, icol = carry
    else:
        m_sum, vacc, iacc = carry

    m = jnp.max(m_sum)
    ii = jnp.min(jnp.where(m_sum == m, biota, _BIG))
    r = ii // 128
    lane = ii % 128
    blk = xs_ref[pl.ds(r, 1)][0]
    hit = (blk == m) & (lane8 == lane)
    s = jnp.min(jnp.where(hit, sub8, _BIG))
    gidx = r * 1024 + s * 128 + lane
    sel = (sub8 == s) & (lane8 == lane)
    blk2 = jnp.where(sel, _NEG, blk)
    xs_ref[pl.ds(r, 1)] = blk2[None]
    newrow = jnp.max(blk2, axis=0, keepdims=True)
    m_sum = jnp.where(rowb == r, newrow, m_sum)
    vacc = jnp.where(flat8 == k, m, vacc)
    iacc = jnp.where(flat8 == k, gidx, iacc)
    if forms:
        kio = lax.broadcasted_iota(jnp.int32, (128, 128), 0)
        vcol = jnp.where(kio == k, m, vcol)
        icol = jnp.where(kio == k, gidx, icol)
        return (m_sum, vacc, iacc, vcol, icol)
    return (m_sum, vacc, iacc)


def _init_summary(xs_ref, nb):
    xs = xs_ref[...]
    m_sum = xs[:, 0, :]
    for s in range(1, 8):
        m_sum = jnp.maximum(m_sum, xs[:, s, :])
    return m_sum


def _topk_units_body(x_ref, vals_ref, idx_ref, vcol_ref, icol_ref, xs_ref):
    nb = x_ref.shape[1]
    xs_ref[...] = x_ref[0]
    m_sum = _init_summary(xs_ref, nb)
    init = (m_sum,
            jnp.full((8, 128), -1.0, jnp.float32),
            jnp.zeros((8, 128), jnp.int32),
            jnp.full((128, 128), -1.0, jnp.float32),
            jnp.zeros((128, 128), jnp.int32))
    step = functools.partial(_ext_step, xs_ref=xs_ref, nb=nb, forms=True)
    _, vacc, iacc, vcol, icol = lax.fori_loop(0, _K, step, init)
    vals_ref[0] = vacc
    idx_ref[0] = iacc
    vcol_ref[0] = vcol
    icol_ref[0] = icol


def _topk_final_body(x_ref, vals_ref, idx_ref, xs_ref, *, kk):
    nb = x_ref.shape[1]
    xs_ref[...] = x_ref[0]
    m_sum = _init_summary(xs_ref, nb)
    init = (m_sum,
            jnp.full((8, 128), -1.0, jnp.float32),
            jnp.zeros((8, 128), jnp.int32))
    step = functools.partial(_ext_step, xs_ref=xs_ref, nb=nb, forms=False)
    _, vacc, iacc = lax.fori_loop(0, kk, step, init)
    vals_ref[0] = vacc
    idx_ref[0] = iacc


def _topk_units(x):
    u, nb = x.shape[0], x.shape[1]
    return pl.pallas_call(
        _topk_units_body,
        grid=(u,),
        in_specs=[pl.BlockSpec((1, nb, 8, 128), lambda i: (i, 0, 0, 0))],
        out_specs=[
            pl.BlockSpec((1, 8, 128), lambda i: (i, 0, 0)),
            pl.BlockSpec((1, 8, 128), lambda i: (i, 0, 0)),
            pl.BlockSpec((1, 128, 128), lambda i: (i, 0, 0)),
            pl.BlockSpec((1, 128, 128), lambda i: (i, 0, 0)),
        ],
        out_shape=[
            jax.ShapeDtypeStruct((u, 8, 128), jnp.float32),
            jax.ShapeDtypeStruct((u, 8, 128), jnp.int32),
            jax.ShapeDtypeStruct((u, 128, 128), jnp.float32),
            jax.ShapeDtypeStruct((u, 128, 128), jnp.int32),
        ],
        scratch_shapes=[pltpu.VMEM((nb, 8, 128), jnp.float32)],
    )(x)


def _topk_final(x, kk):
    nb = x.shape[1]
    return pl.pallas_call(
        functools.partial(_topk_final_body, kk=kk),
        grid=(1,),
        in_specs=[pl.BlockSpec((1, nb, 8, 128), lambda i: (0, 0, 0, 0))],
        out_specs=[
            pl.BlockSpec((1, 8, 128), lambda i: (0, 0, 0)),
            pl.BlockSpec((1, 8, 128), lambda i: (0, 0, 0)),
        ],
        out_shape=[
            jax.ShapeDtypeStruct((1, 8, 128), jnp.float32),
            jax.ShapeDtypeStruct((1, 8, 128), jnp.int32),
        ],
        scratch_shapes=[pltpu.VMEM((nb, 8, 128), jnp.float32)],
    )(x)


def _div80(i):
    return ((i.astype(jnp.float32) + 0.5) * (1.0 / 80.0)).astype(jnp.int32)


def _pairs_body(vcol_ref, icol_ref, vrow_ref, irow_ref, proc_ref,
                keys_ref, ttl_ref, tbr_ref):
    l = pl.program_id(0)
    tl_s = vcol_ref[0][:, 0:1]
    tl_i = icol_ref[0][:, 0:1]
    br_s = vrow_ref[0][0:1, :]
    br_i = irow_ref[0][0:1, :]

    tl_p = _div80(tl_i)
    tl_c = tl_i - tl_p * 80
    br_p = _div80(br_i)
    br_c = br_i - br_p * 80
    tl_y = (tl_p // 64).astype(jnp.float32)
    tl_x = (tl_p % 64).astype(jnp.float32)
    br_y = (br_p // 64).astype(jnp.float32)
    br_x = (br_p % 64).astype(jnp.float32)

    regr = proc_ref[0][:, 2 * _NCLS:2 * _NCLS + 4]
    p_row = lax.broadcasted_iota(jnp.int32, (1, _PP), 1)
    p_col = lax.broadcasted_iota(jnp.int32, (_PP, 1), 0)
    oh_tl = (tl_p == p_row).astype(jnp.float32)
    rt = jnp.dot(oh_tl, regr, preferred_element_type=jnp.float32)
    oh_brT = (p_col == br_p).astype(jnp.float32)
    rb = lax.dot_general(regr, oh_brT, (((0,), (0,)), ((), ())),
                         preferred_element_type=jnp.float32)

    tlx = tl_x + rt[:, 0:1]
    tly = tl_y + rt[:, 1:2]
    brx = br_x + rb[2:3, :]
    bry = br_y + rb[3:4, :]

    i_io = lax.broadcasted_iota(jnp.int32, (128, 1), 0)
    j_io = lax.broadcasted_iota(jnp.int32, (1, 128), 1)
    in_k = (i_io < _K) & (j_io < _K)
    score = (tl_s + br_s) * 0.5
    bad = (tl_c != br_c) | (brx < tlx) | (bry < tly)
    semidx = (l * 10000 + i_io * 100 + j_io).astype(jnp.float32)
    inv_key = -1.0 - semidx * (2.0 ** -14)
    keys = jnp.where(in_k, jnp.where(bad, inv_key, score), _NEG)
    keys_ref[0] = keys

    z = jnp.zeros((128, 1), jnp.float32)
    ttl_ref[0] = jnp.concatenate([tlx, tly] + [z] * 6, axis=1)
    zr = jnp.zeros((1, 128), jnp.float32)
    tbr_ref[0] = jnp.concatenate([zr, zr, brx, bry] + [zr] * 4, axis=0)


def _pairs(vcol, icol, uvals, uidx, proc):
    return pl.pallas_call(
        _pairs_body,
        grid=(_NL,),
        in_specs=[
            pl.BlockSpec((1, 128, 128), lambda l: (l, 0, 0)),
            pl.BlockSpec((1, 128, 128), lambda l: (l, 0, 0)),
            pl.BlockSpec((1, 8, 128), lambda l: (l + _NL, 0, 0)),
            pl.BlockSpec((1, 8, 128), lambda l: (l + _NL, 0, 0)),
            pl.BlockSpec((1, _PP, _CF), lambda l: (l, 0, 0)),
        ],
        out_specs=[
            pl.BlockSpec((1, 128, 128), lambda l: (l, 0, 0)),
            pl.BlockSpec((1, 128, 8), lambda l: (l, 0, 0)),
            pl.BlockSpec((1, 8, 128), lambda l: (l, 0, 0)),
        ],
        out_shape=[
            jax.ShapeDtypeStruct((_NL, 128, 128), jnp.float32),
            jax.ShapeDtypeStruct((_NL, 128, 8), jnp.float32),
            jax.ShapeDtypeStruct((_NL, 8, 128), jnp.float32),
        ],
    )(vcol, icol, uvals, uidx, proc)


def _gather_body(idxc_ref, valc_ref, tbl_ref, out_ref):
    f = idxc_ref[...]
    v = valc_ref[...]
    l = f // 16384
    rem = f - l * 16384
    i = rem // 128
    j = rem - i * 128
    li = l * 128 + i
    lj = l * 128 + j
    col = lax.broadcasted_iota(jnp.int32, (1, _NL * 128), 1)
    oh_li = (li == col).astype(jnp.float32)
    oh_lj = (lj == col).astype(jnp.float32)
    tbl = tbl_ref[...]
    a = jnp.dot(oh_li, tbl, preferred_element_type=jnp.float32)
    b = jnp.dot(oh_lj, tbl, preferred_element_type=jnp.float32)
    sc = jnp.where(v < 0.0, -1.0, v)
    z = jnp.zeros((1024, 1), jnp.float32)
    out_ref[...] = jnp.concatenate(
        [a[:, 0:2], b[:, 2:4], sc, z, z, z], axis=1)


def _gatherE(idx_col, val_col, tbl):
    return pl.pallas_call(
        _gather_body,
        out_shape=jax.ShapeDtypeStruct((1024, 8), jnp.float32),
    )(idx_col, val_col, tbl)


@jax.jit
def _run(feats, w_all):
    proc = _heads(feats, w_all)
    tl = proc[:, :, :_NCLS].reshape(_NL, 320, 8, 128)
    br = proc[:, :, _NCLS:2 * _NCLS].reshape(_NL, 320, 8, 128)
    units = jnp.concatenate([tl, br], axis=0)
    uvals, uidx, uvcol, uicol = _topk_units(units)
    keys, ttl, tbr = _pairs(uvcol, uicol, uvals, uidx, proc)
    fvals, fidx = _topk_final(keys.reshape(1, _NL * 16, 8, 128), 1000)
    idx_col = fidx.reshape(1024)[:, None]
    val_col = fvals.reshape(1024)[:, None]
    tblc = jnp.concatenate(
        [ttl.reshape(_NL * 128, 8)[:, :2],
         jnp.transpose(tbr, (0, 2, 1)).reshape(_NL * 128, 8)[:, 2:4],
         jnp.zeros((_NL * 128, 4), jnp.float32)], axis=1)
    out = _gatherE(idx_col, val_col, tblc)
    det_scores = out[:1000, 4][None]
    det_boxes = out[:1000, :4][None]
    return det_scores, det_boxes


def kernel(feat_0, feat_1, feat_2, feat_3, feat_4, feat_5, feat_6,
           W_tl_heat, W_br_heat, W_tl_regr, W_br_regr):
    feats_in = [feat_0, feat_1, feat_2, feat_3, feat_4, feat_5, feat_6]
    padded = []
    for f, (h, w) in zip(feats_in, _SIZES):
        fm = jnp.transpose(f[0], (1, 2, 0))
        fm = jnp.pad(fm, ((0, _PH - h), (0, _PW - w), (0, 0)))
        padded.append(fm.reshape(_PP, _CF))
    feats = jnp.stack(padded, axis=0)
    w_all = jnp.concatenate(
        [W_tl_heat, W_br_heat, W_tl_regr, W_br_regr,
         jnp.zeros((_CF, _CF - 2 * _NCLS - 4), jnp.float32)], axis=1)
    return _run(feats, w_all)


# speedup vs baseline: 1.4725x; 1.0287x over previous
import functools
import jax
import jax.numpy as jnp
from jax import lax
from jax.experimental import pallas as pl
from jax.experimental.pallas import tpu as pltpu

_K = 100
_NCLS = 80
_CF = 256
_SIZES = [(64, 64), (64, 32), (32, 64), (32, 32), (32, 16), (16, 32), (16, 16)]
_NL = len(_SIZES)
_PH = 64
_PW = 64
_PP = _PH * _PW
_NEG = -1e9
_BIG = 2 ** 30


def _head_nms_body(x_ref, w_ref, o_ref):
    l = pl.program_id(0)
    x = x_ref[0]
    w = w_ref[...]
    logits = jnp.dot(x, w, preferred_element_type=jnp.float32)

    heat = jnp.clip(jax.nn.sigmoid(logits[:, :2 * _NCLS]), 1e-4, 1.0 - 1e-4)

    h_l = sum((l == i) * h for i, (h, _) in enumerate(_SIZES))
    w_l = sum((l == i) * w_ for i, (_, w_) in enumerate(_SIZES))

    r = lax.broadcasted_iota(jnp.int32, (_PP, 1), 0)
    yy = r // _PW
    xx = r % _PW
    valid = (yy < h_l) & (xx < w_l)

    hv = jnp.where(valid, heat, _NEG)

    neg_row = jnp.full((1, 2 * _NCLS), _NEG, jnp.float32)
    left = jnp.where(xx == 0, _NEG,
                     jnp.concatenate([neg_row, hv[:-1, :]], axis=0))
    right = jnp.where(xx == _PW - 1, _NEG,
                      jnp.concatenate([hv[1:, :], neg_row], axis=0))
    hx = jnp.maximum(hv, jnp.maximum(left, right))
    neg_blk = jnp.full((_PW, 2 * _NCLS), _NEG, jnp.float32)
    up = jnp.concatenate([neg_blk, hx[:-_PW, :]], axis=0)
    dn = jnp.concatenate([hx[_PW:, :], neg_blk], axis=0)
    hmax = jnp.maximum(hx, jnp.maximum(up, dn))

    keep = (hmax == hv).astype(jnp.float32)
    scores = jnp.where(valid, heat * keep, 0.0)

    o_ref[0] = jnp.concatenate([scores, logits[:, 2 * _NCLS:]], axis=1)


def _heads(feats, w_all):
    return pl.pallas_call(
        _head_nms_body,
        grid=(_NL,),
        in_specs=[
            pl.BlockSpec((1, _PP, _CF), lambda l: (l, 0, 0)),
            pl.BlockSpec((_CF, _CF), lambda l: (0, 0)),
        ],
        out_specs=pl.BlockSpec((1, _PP, _CF), lambda l: (l, 0, 0)),
        out_shape=jax.ShapeDtypeStruct((_NL, _PP, _CF), jnp.float32),
    )(feats, w_all)


def _ext_step(k, carry, xs_ref, nb, forms):
    sub8 = lax.broadcasted_iota(jnp.int32, (8, 128), 0)
    lane8 = lax.broadcasted_iota(jnp.int32, (8, 128), 1)
    flat8 = sub8 * 128 + lane8
    biota = (lax.broadcasted_iota(jnp.int32, (nb, 128), 0) * 128 +
             lax.broadcasted_iota(jnp.int32, (nb, 128), 1))
    rowb = lax.broadcasted_iota(jnp.int32, (nb, 1), 0)

    m_sum, vacc, iacc = carry

    m = jnp.max(m_sum)
    ii = jnp.min(jnp.where(m_sum == m, biota, _BIG))
    r = ii // 128
    lane = ii % 128
    blk = xs_ref[pl.ds(r, 1)][0]
    hit = (blk == m) & (lane8 == lane)
    s = jnp.min(jnp.where(hit, sub8, _BIG))
    gidx = r * 1024 + s * 128 + lane
    sel = (sub8 == s) & (lane8 == lane)
    blk2 = jnp.where(sel, _NEG, blk)
    xs_ref[pl.ds(r, 1)] = blk2[None]
    newrow = jnp.max(blk2, axis=0, keepdims=True)
    m_sum = jnp.where(rowb == r, newrow, m_sum)
    vacc = jnp.where(flat8 == k, m, vacc)
    iacc = jnp.where(flat8 == k, gidx, iacc)
    return (m_sum, vacc, iacc)


def _init_summary(xs_ref, nb):
    xs = xs_ref[...]
    m_sum = xs[:, 0, :]
    for s in range(1, 8):
        m_sum = jnp.maximum(m_sum, xs[:, s, :])
    return m_sum


def _run_extraction(x_ref, xs_ref, kk):
    nb = x_ref.shape[1]
    xs_ref[...] = x_ref[0]
    m_sum = _init_summary(xs_ref, nb)
    init = (m_sum,
            jnp.full((8, 128), -1.0, jnp.float32),
            jnp.zeros((8, 128), jnp.int32))
    step = functools.partial(_ext_step, xs_ref=xs_ref, nb=nb, forms=False)

    def step2(t, c):
        return step(2 * t + 1, step(2 * t, c))

    _, vacc, iacc = lax.fori_loop(0, kk // 2, step2, init)
    return vacc, iacc


def _topk_units_body(x_ref, vals_ref, idx_ref, vcol_ref, icol_ref, xs_ref):
    vacc, iacc = _run_extraction(x_ref, xs_ref, _K)
    vals_ref[0] = vacc
    idx_ref[0] = iacc
    io0 = lax.broadcasted_iota(jnp.int32, (128, 128), 0)
    io1 = lax.broadcasted_iota(jnp.int32, (128, 128), 1)
    eye = (io0 == io1).astype(jnp.float32)
    vt = lax.dot_general(eye, vacc[0:1, :], (((1,), (1,)), ((), ())),
                         preferred_element_type=jnp.float32)
    it = lax.dot_general(eye, iacc[0:1, :].astype(jnp.float32),
                         (((1,), (1,)), ((), ())),
                         preferred_element_type=jnp.float32)
    vcol_ref[0] = jnp.broadcast_to(vt, (128, 128))
    icol_ref[0] = jnp.broadcast_to(it.astype(jnp.int32), (128, 128))


def _topk_final_body(x_ref, vals_ref, idx_ref, xs_ref, *, kk):
    vacc, iacc = _run_extraction(x_ref, xs_ref, kk)
    vals_ref[0] = vacc
    idx_ref[0] = iacc


def _topk_units(x):
    u, nb = x.shape[0], x.shape[1]
    return pl.pallas_call(
        _topk_units_body,
        grid=(u,),
        in_specs=[pl.BlockSpec((1, nb, 8, 128), lambda i: (i, 0, 0, 0))],
        out_specs=[
            pl.BlockSpec((1, 8, 128), lambda i: (i, 0, 0)),
            pl.BlockSpec((1, 8, 128), lambda i: (i, 0, 0)),
            pl.BlockSpec((1, 128, 128), lambda i: (i, 0, 0)),
            pl.BlockSpec((1, 128, 128), lambda i: (i, 0, 0)),
        ],
        out_shape=[
            jax.ShapeDtypeStruct((u, 8, 128), jnp.float32),
            jax.ShapeDtypeStruct((u, 8, 128), jnp.int32),
            jax.ShapeDtypeStruct((u, 128, 128), jnp.float32),
            jax.ShapeDtypeStruct((u, 128, 128), jnp.int32),
        ],
        scratch_shapes=[pltpu.VMEM((nb, 8, 128), jnp.float32)],
    )(x)


def _topk_final(x, kk):
    nb = x.shape[1]
    return pl.pallas_call(
        functools.partial(_topk_final_body, kk=kk),
        grid=(1,),
        in_specs=[pl.BlockSpec((1, nb, 8, 128), lambda i: (0, 0, 0, 0))],
        out_specs=[
            pl.BlockSpec((1, 8, 128), lambda i: (0, 0, 0)),
            pl.BlockSpec((1, 8, 128), lambda i: (0, 0, 0)),
        ],
        out_shape=[
            jax.ShapeDtypeStruct((1, 8, 128), jnp.float32),
            jax.ShapeDtypeStruct((1, 8, 128), jnp.int32),
        ],
        scratch_shapes=[pltpu.VMEM((nb, 8, 128), jnp.float32)],
    )(x)


def _div80(i):
    return ((i.astype(jnp.float32) + 0.5) * (1.0 / 80.0)).astype(jnp.int32)


def _pairs_body(vcol_ref, icol_ref, vrow_ref, irow_ref, proc_ref,
                keys_ref, ttl_ref, tbr_ref):
    l = pl.program_id(0)
    tl_s = vcol_ref[0][:, 0:1]
    tl_i = icol_ref[0][:, 0:1]
    br_s = vrow_ref[0][0:1, :]
    br_i = irow_ref[0][0:1, :]

    tl_p = _div80(tl_i)
    tl_c = tl_i - tl_p * 80
    br_p = _div80(br_i)
    br_c = br_i - br_p * 80
    tl_y = (tl_p // 64).astype(jnp.float32)
    tl_x = (tl_p % 64).astype(jnp.float32)
    br_y = (br_p // 64).astype(jnp.float32)
    br_x = (br_p % 64).astype(jnp.float32)

    regr = proc_ref[0][:, 2 * _NCLS:2 * _NCLS + 4]
    p_row = lax.broadcasted_iota(jnp.int32, (1, _PP), 1)
    p_col = lax.broadcasted_iota(jnp.int32, (_PP, 1), 0)
    oh_tl = (tl_p == p_row).astype(jnp.float32)
    rt = jnp.dot(oh_tl, regr, preferred_element_type=jnp.float32)
    oh_brT = (p_col == br_p).astype(jnp.float32)
    rb = lax.dot_general(regr, oh_brT, (((0,), (0,)), ((), ())),
                         preferred_element_type=jnp.float32)

    tlx = tl_x + rt[:, 0:1]
    tly = tl_y + rt[:, 1:2]
    brx = br_x + rb[2:3, :]
    bry = br_y + rb[3:4, :]

    i_io = lax.broadcasted_iota(jnp.int32, (128, 1), 0)
    j_io = lax.broadcasted_iota(jnp.int32, (1, 128), 1)
    in_k = (i_io < _K) & (j_io < _K)
    score = (tl_s + br_s) * 0.5
    bad = (tl_c != br_c) | (brx < tlx) | (bry < tly)
    semidx = (l * 10000 + i_io * 100 + j_io).astype(jnp.float32)
    inv_key = -1.0 - semidx * (2.0 ** -14)
    keys = jnp.where(in_k, jnp.where(bad, inv_key, score), _NEG)
    keys_ref[0] = keys

    z = jnp.zeros((128, 1), jnp.float32)
    ttl_ref[0] = jnp.concatenate([tlx, tly] + [z] * 6, axis=1)
    zr = jnp.zeros((1, 128), jnp.float32)
    tbr_ref[0] = jnp.concatenate([zr, zr, brx, bry] + [zr] * 4, axis=0)


def _pairs(vcol, icol, uvals, uidx, proc):
    return pl.pallas_call(
        _pairs_body,
        grid=(_NL,),
        in_specs=[
            pl.BlockSpec((1, 128, 128), lambda l: (l, 0, 0)),
            pl.BlockSpec((1, 128, 128), lambda l: (l, 0, 0)),
            pl.BlockSpec((1, 8, 128), lambda l: (l + _NL, 0, 0)),
            pl.BlockSpec((1, 8, 128), lambda l: (l + _NL, 0, 0)),
            pl.BlockSpec((1, _PP, _CF), lambda l: (l, 0, 0)),
        ],
        out_specs=[
            pl.BlockSpec((1, 128, 128), lambda l: (l, 0, 0)),
            pl.BlockSpec((1, 128, 8), lambda l: (l, 0, 0)),
            pl.BlockSpec((1, 8, 128), lambda l: (l, 0, 0)),
        ],
        out_shape=[
            jax.ShapeDtypeStruct((_NL, 128, 128), jnp.float32),
            jax.ShapeDtypeStruct((_NL, 128, 8), jnp.float32),
            jax.ShapeDtypeStruct((_NL, 8, 128), jnp.float32),
        ],
    )(vcol, icol, uvals, uidx, proc)


def _gather_body(idxc_ref, valc_ref, tbl_ref, out_ref):
    f = idxc_ref[...]
    v = valc_ref[...]
    l = f // 16384
    rem = f - l * 16384
    i = rem // 128
    j = rem - i * 128
    li = l * 128 + i
    lj = l * 128 + j
    col = lax.broadcasted_iota(jnp.int32, (1, _NL * 128), 1)
    oh_li = (li == col).astype(jnp.float32)
    oh_lj = (lj == col).astype(jnp.float32)
    tbl = tbl_ref[...]
    a = jnp.dot(oh_li, tbl, preferred_element_type=jnp.float32)
    b = jnp.dot(oh_lj, tbl, preferred_element_type=jnp.float32)
    sc = jnp.where(v < 0.0, -1.0, v)
    z = jnp.zeros((1024, 1), jnp.float32)
    out_ref[...] = jnp.concatenate(
        [a[:, 0:2], b[:, 2:4], sc, z, z, z], axis=1)


def _gatherE(idx_col, val_col, tbl):
    return pl.pallas_call(
        _gather_body,
        out_shape=jax.ShapeDtypeStruct((1024, 8), jnp.float32),
    )(idx_col, val_col, tbl)


@jax.jit
def _run(feats, w_all):
    proc = _heads(feats, w_all)
    tl = proc[:, :, :_NCLS].reshape(_NL, 320, 8, 128)
    br = proc[:, :, _NCLS:2 * _NCLS].reshape(_NL, 320, 8, 128)
    units = jnp.concatenate([tl, br], axis=0)
    uvals, uidx, uvcol, uicol = _topk_units(units)
    keys, ttl, tbr = _pairs(uvcol, uicol, uvals, uidx, proc)
    fvals, fidx = _topk_final(keys.reshape(1, _NL * 16, 8, 128), 1000)
    idx_col = fidx.reshape(1024)[:, None]
    val_col = fvals.reshape(1024)[:, None]
    tblc = jnp.concatenate(
        [ttl.reshape(_NL * 128, 8)[:, :2],
         jnp.transpose(tbr, (0, 2, 1)).reshape(_NL * 128, 8)[:, 2:4],
         jnp.zeros((_NL * 128, 4), jnp.float32)], axis=1)
    out = _gatherE(idx_col, val_col, tblc)
    det_scores = out[:1000, 4][None]
    det_boxes = out[:1000, :4][None]
    return det_scores, det_boxes


def kernel(feat_0, feat_1, feat_2, feat_3, feat_4, feat_5, feat_6,
           W_tl_heat, W_br_heat, W_tl_regr, W_br_regr):
    feats_in = [feat_0, feat_1, feat_2, feat_3, feat_4, feat_5, feat_6]
    padded = []
    for f, (h, w) in zip(feats_in, _SIZES):
        fm = jnp.transpose(f[0], (1, 2, 0))
        fm = jnp.pad(fm, ((0, _PH - h), (0, _PW - w), (0, 0)))
        padded.append(fm.reshape(_PP, _CF))
    feats = jnp.stack(padded, axis=0)
    w_all = jnp.concatenate(
        [W_tl_heat, W_br_heat, W_tl_regr, W_br_regr,
         jnp.zeros((_CF, _CF - 2 * _NCLS - 4), jnp.float32)], axis=1)
    return _run(feats, w_all)


# speedup vs baseline: 1.6575x; 1.1256x over previous
import functools
import jax
import jax.numpy as jnp
from jax import lax
from jax.experimental import pallas as pl
from jax.experimental.pallas import tpu as pltpu

_K = 100
_NCLS = 80
_CF = 256
_SIZES = [(64, 64), (64, 32), (32, 64), (32, 32), (32, 16), (16, 32), (16, 16)]
_NL = len(_SIZES)
_PH = 64
_PW = 64
_PP = _PH * _PW
_NEG = -1e9
_BIG = 2 ** 30


def _head_nms_body(x_ref, w_ref, o_ref):
    l = pl.program_id(0)
    x = x_ref[0]
    w = w_ref[...]
    logits = jnp.dot(x, w, preferred_element_type=jnp.float32)

    heat = jnp.clip(jax.nn.sigmoid(logits[:, :2 * _NCLS]), 1e-4, 1.0 - 1e-4)

    h_l = sum((l == i) * h for i, (h, _) in enumerate(_SIZES))
    w_l = sum((l == i) * w_ for i, (_, w_) in enumerate(_SIZES))

    r = lax.broadcasted_iota(jnp.int32, (_PP, 1), 0)
    yy = r // _PW
    xx = r % _PW
    valid = (yy < h_l) & (xx < w_l)

    hv = jnp.where(valid, heat, _NEG)

    neg_row = jnp.full((1, 2 * _NCLS), _NEG, jnp.float32)
    left = jnp.where(xx == 0, _NEG,
                     jnp.concatenate([neg_row, hv[:-1, :]], axis=0))
    right = jnp.where(xx == _PW - 1, _NEG,
                      jnp.concatenate([hv[1:, :], neg_row], axis=0))
    hx = jnp.maximum(hv, jnp.maximum(left, right))
    neg_blk = jnp.full((_PW, 2 * _NCLS), _NEG, jnp.float32)
    up = jnp.concatenate([neg_blk, hx[:-_PW, :]], axis=0)
    dn = jnp.concatenate([hx[_PW:, :], neg_blk], axis=0)
    hmax = jnp.maximum(hx, jnp.maximum(up, dn))

    keep = (hmax == hv).astype(jnp.float32)
    scores = jnp.where(valid, heat * keep, 0.0)

    o_ref[0] = jnp.concatenate([scores, logits[:, 2 * _NCLS:]], axis=1)


def _heads(feats, w_all):
    return pl.pallas_call(
        _head_nms_body,
        grid=(_NL,),
        in_specs=[
            pl.BlockSpec((1, _PP, _CF), lambda l: (l, 0, 0)),
            pl.BlockSpec((_CF, _CF), lambda l: (0, 0)),
        ],
        out_specs=pl.BlockSpec((1, _PP, _CF), lambda l: (l, 0, 0)),
        out_shape=jax.ShapeDtypeStruct((_NL, _PP, _CF), jnp.float32),
    )(feats, w_all)


def _ext_step(k, carry, xs_ref, nb, forms):
    sub8 = lax.broadcasted_iota(jnp.int32, (8, 128), 0)
    lane8 = lax.broadcasted_iota(jnp.int32, (8, 128), 1)
    flat8 = sub8 * 128 + lane8
    biota = (lax.broadcasted_iota(jnp.int32, (nb, 128), 0) * 128 +
             lax.broadcasted_iota(jnp.int32, (nb, 128), 1))
    rowb = lax.broadcasted_iota(jnp.int32, (nb, 1), 0)

    m_sum, vacc, iacc = carry

    m = jnp.max(m_sum)
    ii = jnp.min(jnp.where(m_sum == m, biota, _BIG))
    r = ii // 128
    lane = ii % 128
    blk = xs_ref[pl.ds(r, 1)][0]
    hit = (blk == m) & (lane8 == lane)
    s = jnp.min(jnp.where(hit, sub8, _BIG))
    gidx = r * 1024 + s * 128 + lane
    sel = (sub8 == s) & (lane8 == lane)
    blk2 = jnp.where(sel, _NEG, blk)
    xs_ref[pl.ds(r, 1)] = blk2[None]
    newrow = jnp.max(blk2, axis=0, keepdims=True)
    m_sum = jnp.where(rowb == r, newrow, m_sum)
    vacc = jnp.where(flat8 == k, m, vacc)
    iacc = jnp.where(flat8 == k, gidx, iacc)
    return (m_sum, vacc, iacc)


def _init_summary(xs_ref, nb):
    xs = xs_ref[...]
    m_sum = xs[:, 0, :]
    for s in range(1, 8):
        m_sum = jnp.maximum(m_sum, xs[:, s, :])
    return m_sum


def _run_extraction(x_ref, xs_ref, kk):
    nb = x_ref.shape[1]
    xs_ref[...] = x_ref[0]
    m_sum = _init_summary(xs_ref, nb)
    init = (m_sum,
            jnp.full((8, 128), -1.0, jnp.float32),
            jnp.zeros((8, 128), jnp.int32))
    step = functools.partial(_ext_step, xs_ref=xs_ref, nb=nb, forms=False)

    def step2(t, c):
        return step(2 * t + 1, step(2 * t, c))

    _, vacc, iacc = lax.fori_loop(0, kk // 2, step2, init)
    return vacc, iacc


def _unit_step(k, carry, xs_ref, u, nb):
    sub8 = lax.broadcasted_iota(jnp.int32, (8, 128), 0)
    lane8 = lax.broadcasted_iota(jnp.int32, (8, 128), 1)
    flat8 = sub8 * 128 + lane8
    biota = (lax.broadcasted_iota(jnp.int32, (nb, 128), 0) * 128 +
             lax.broadcasted_iota(jnp.int32, (nb, 128), 1))
    rowb = lax.broadcasted_iota(jnp.int32, (nb, 1), 0)

    m_sum, vacc, iacc = carry
    m = jnp.max(m_sum)
    ii = jnp.min(jnp.where(m_sum == m, biota, _BIG))
    r = ii // 128
    lane = ii % 128
    blk = xs_ref[u, pl.ds(r, 1)][0]
    hit = (blk == m) & (lane8 == lane)
    s = jnp.min(jnp.where(hit, sub8, _BIG))
    gidx = r * 1024 + s * 128 + lane
    sel = (sub8 == s) & (lane8 == lane)
    blk2 = jnp.where(sel, _NEG, blk)
    xs_ref[u, pl.ds(r, 1)] = blk2[None]
    newrow = jnp.max(blk2, axis=0, keepdims=True)
    m_sum = jnp.where(rowb == r, newrow, m_sum)
    vacc = jnp.where(flat8 == k, m, vacc)
    iacc = jnp.where(flat8 == k, gidx, iacc)
    return (m_sum, vacc, iacc)


def _topk_units_body(x_ref, vals_ref, idx_ref, vcol_ref, icol_ref, xs_ref):
    nu, nb = x_ref.shape[0], x_ref.shape[1]
    xs_ref[...] = x_ref[...]
    msums, vaccs, iaccs = [], [], []
    for u in range(nu):
        xu = xs_ref[u]
        m = xu[:, 0, :]
        for s in range(1, 8):
            m = jnp.maximum(m, xu[:, s, :])
        msums.append(m)
        vaccs.append(jnp.full((8, 128), -1.0, jnp.float32))
        iaccs.append(jnp.zeros((8, 128), jnp.int32))

    def step(k, carry):
        ms, vs, is_ = carry
        out = [_unit_step(k, (ms[u], vs[u], is_[u]), xs_ref, u, nb)
               for u in range(nu)]
        return (tuple(o[0] for o in out), tuple(o[1] for o in out),
                tuple(o[2] for o in out))

    def step2(t, c):
        return step(2 * t + 1, step(2 * t, c))

    _, vaccs, iaccs = lax.fori_loop(
        0, _K // 2, step2, (tuple(msums), tuple(vaccs), tuple(iaccs)))

    io0 = lax.broadcasted_iota(jnp.int32, (128, 128), 0)
    io1 = lax.broadcasted_iota(jnp.int32, (128, 128), 1)
    eye = (io0 == io1).astype(jnp.float32)
    for u in range(nu):
        vals_ref[u] = vaccs[u]
        idx_ref[u] = iaccs[u]
        vt = lax.dot_general(eye, vaccs[u][0:1, :], (((1,), (1,)), ((), ())),
                             preferred_element_type=jnp.float32)
        it = lax.dot_general(eye, iaccs[u][0:1, :].astype(jnp.float32),
                             (((1,), (1,)), ((), ())),
                             preferred_element_type=jnp.float32)
        vcol_ref[u] = jnp.broadcast_to(vt, (128, 128))
        icol_ref[u] = jnp.broadcast_to(it.astype(jnp.int32), (128, 128))


def _topk_final_body(x_ref, vals_ref, idx_ref, xs_ref, *, kk):
    vacc, iacc = _run_extraction(x_ref, xs_ref, kk)
    vals_ref[0] = vacc
    idx_ref[0] = iacc


def _topk_units(x):
    u, nb = x.shape[0], x.shape[1]
    return pl.pallas_call(
        _topk_units_body,
        out_shape=[
            jax.ShapeDtypeStruct((u, 8, 128), jnp.float32),
            jax.ShapeDtypeStruct((u, 8, 128), jnp.int32),
            jax.ShapeDtypeStruct((u, 128, 128), jnp.float32),
            jax.ShapeDtypeStruct((u, 128, 128), jnp.int32),
        ],
        scratch_shapes=[pltpu.VMEM((u, nb, 8, 128), jnp.float32)],
    )(x)


def _topk_final(x, kk):
    nb = x.shape[1]
    return pl.pallas_call(
        functools.partial(_topk_final_body, kk=kk),
        grid=(1,),
        in_specs=[pl.BlockSpec((1, nb, 8, 128), lambda i: (0, 0, 0, 0))],
        out_specs=[
            pl.BlockSpec((1, 8, 128), lambda i: (0, 0, 0)),
            pl.BlockSpec((1, 8, 128), lambda i: (0, 0, 0)),
        ],
        out_shape=[
            jax.ShapeDtypeStruct((1, 8, 128), jnp.float32),
            jax.ShapeDtypeStruct((1, 8, 128), jnp.int32),
        ],
        scratch_shapes=[pltpu.VMEM((nb, 8, 128), jnp.float32)],
    )(x)


def _div80(i):
    return ((i.astype(jnp.float32) + 0.5) * (1.0 / 80.0)).astype(jnp.int32)


def _pairs_body(vcol_ref, icol_ref, vrow_ref, irow_ref, proc_ref,
                keys_ref, ttl_ref, tbr_ref):
    l = pl.program_id(0)
    tl_s = vcol_ref[0][:, 0:1]
    tl_i = icol_ref[0][:, 0:1]
    br_s = vrow_ref[0][0:1, :]
    br_i = irow_ref[0][0:1, :]

    tl_p = _div80(tl_i)
    tl_c = tl_i - tl_p * 80
    br_p = _div80(br_i)
    br_c = br_i - br_p * 80
    tl_y = (tl_p // 64).astype(jnp.float32)
    tl_x = (tl_p % 64).astype(jnp.float32)
    br_y = (br_p // 64).astype(jnp.float32)
    br_x = (br_p % 64).astype(jnp.float32)

    regr = proc_ref[0][:, 2 * _NCLS:2 * _NCLS + 4]
    p_row = lax.broadcasted_iota(jnp.int32, (1, _PP), 1)
    p_col = lax.broadcasted_iota(jnp.int32, (_PP, 1), 0)
    oh_tl = (tl_p == p_row).astype(jnp.float32)
    rt = jnp.dot(oh_tl, regr, preferred_element_type=jnp.float32)
    oh_brT = (p_col == br_p).astype(jnp.float32)
    rb = lax.dot_general(regr, oh_brT, (((0,), (0,)), ((), ())),
                         preferred_element_type=jnp.float32)

    tlx = tl_x + rt[:, 0:1]
    tly = tl_y + rt[:, 1:2]
    brx = br_x + rb[2:3, :]
    bry = br_y + rb[3:4, :]

    i_io = lax.broadcasted_iota(jnp.int32, (128, 1), 0)
    j_io = lax.broadcasted_iota(jnp.int32, (1, 128), 1)
    in_k = (i_io < _K) & (j_io < _K)
    score = (tl_s + br_s) * 0.5
    bad = (tl_c != br_c) | (brx < tlx) | (bry < tly)
    semidx = (l * 10000 + i_io * 100 + j_io).astype(jnp.float32)
    inv_key = -1.0 - semidx * (2.0 ** -14)
    keys = jnp.where(in_k, jnp.where(bad, inv_key, score), _NEG)
    keys_ref[0] = keys

    z = jnp.zeros((128, 1), jnp.float32)
    ttl_ref[0] = jnp.concatenate([tlx, tly] + [z] * 6, axis=1)
    zr = jnp.zeros((1, 128), jnp.float32)
    tbr_ref[0] = jnp.concatenate([zr, zr, brx, bry] + [zr] * 4, axis=0)


def _pairs(vcol, icol, uvals, uidx, proc):
    return pl.pallas_call(
        _pairs_body,
        grid=(_NL,),
        in_specs=[
            pl.BlockSpec((1, 128, 128), lambda l: (l, 0, 0)),
            pl.BlockSpec((1, 128, 128), lambda l: (l, 0, 0)),
            pl.BlockSpec((1, 8, 128), lambda l: (l + _NL, 0, 0)),
            pl.BlockSpec((1, 8, 128), lambda l: (l + _NL, 0, 0)),
            pl.BlockSpec((1, _PP, _CF), lambda l: (l, 0, 0)),
        ],
        out_specs=[
            pl.BlockSpec((1, 128, 128), lambda l: (l, 0, 0)),
            pl.BlockSpec((1, 128, 8), lambda l: (l, 0, 0)),
            pl.BlockSpec((1, 8, 128), lambda l: (l, 0, 0)),
        ],
        out_shape=[
            jax.ShapeDtypeStruct((_NL, 128, 128), jnp.float32),
            jax.ShapeDtypeStruct((_NL, 128, 8), jnp.float32),
            jax.ShapeDtypeStruct((_NL, 8, 128), jnp.float32),
        ],
    )(vcol, icol, uvals, uidx, proc)


def _gather_body(idxc_ref, valc_ref, tbl_ref, out_ref):
    f = idxc_ref[...]
    v = valc_ref[...]
    l = f // 16384
    rem = f - l * 16384
    i = rem // 128
    j = rem - i * 128
    li = l * 128 + i
    lj = l * 128 + j
    col = lax.broadcasted_iota(jnp.int32, (1, _NL * 128), 1)
    oh_li = (li == col).astype(jnp.float32)
    oh_lj = (lj == col).astype(jnp.float32)
    tbl = tbl_ref[...]
    a = jnp.dot(oh_li, tbl, preferred_element_type=jnp.float32)
    b = jnp.dot(oh_lj, tbl, preferred_element_type=jnp.float32)
    sc = jnp.where(v < 0.0, -1.0, v)
    z = jnp.zeros((1024, 1), jnp.float32)
    out_ref[...] = jnp.concatenate(
        [a[:, 0:2], b[:, 2:4], sc, z, z, z], axis=1)


def _gatherE(idx_col, val_col, tbl):
    return pl.pallas_call(
        _gather_body,
        out_shape=jax.ShapeDtypeStruct((1024, 8), jnp.float32),
    )(idx_col, val_col, tbl)


@jax.jit
def _run(feats, w_all):
    proc = _heads(feats, w_all)
    tl = proc[:, :, :_NCLS].reshape(_NL, 320, 8, 128)
    br = proc[:, :, _NCLS:2 * _NCLS].reshape(_NL, 320, 8, 128)
    units = jnp.concatenate([tl, br], axis=0)
    uvals, uidx, uvcol, uicol = _topk_units(units)
    keys, ttl, tbr = _pairs(uvcol, uicol, uvals, uidx, proc)
    fvals, fidx = _topk_final(keys.reshape(1, _NL * 16, 8, 128), 1000)
    idx_col = fidx.reshape(1024)[:, None]
    val_col = fvals.reshape(1024)[:, None]
    tblc = jnp.concatenate(
        [ttl.reshape(_NL * 128, 8)[:, :2],
         jnp.transpose(tbr, (0, 2, 1)).reshape(_NL * 128, 8)[:, 2:4],
         jnp.zeros((_NL * 128, 4), jnp.float32)], axis=1)
    out = _gatherE(idx_col, val_col, tblc)
    det_scores = out[:1000, 4][None]
    det_boxes = out[:1000, :4][None]
    return det_scores, det_boxes


def kernel(feat_0, feat_1, feat_2, feat_3, feat_4, feat_5, feat_6,
           W_tl_heat, W_br_heat, W_tl_regr, W_br_regr):
    feats_in = [feat_0, feat_1, feat_2, feat_3, feat_4, feat_5, feat_6]
    padded = []
    for f, (h, w) in zip(feats_in, _SIZES):
        fm = jnp.transpose(f[0], (1, 2, 0))
        fm = jnp.pad(fm, ((0, _PH - h), (0, _PW - w), (0, 0)))
        padded.append(fm.reshape(_PP, _CF))
    feats = jnp.stack(padded, axis=0)
    w_all = jnp.concatenate(
        [W_tl_heat, W_br_heat, W_tl_regr, W_br_regr,
         jnp.zeros((_CF, _CF - 2 * _NCLS - 4), jnp.float32)], axis=1)
    return _run(feats, w_all)


# speedup vs baseline: 1.6677x; 1.0062x over previous
import functools
import jax
import jax.numpy as jnp
from jax import lax
from jax.experimental import pallas as pl
from jax.experimental.pallas import tpu as pltpu

_K = 100
_NCLS = 80
_CF = 256
_SIZES = [(64, 64), (64, 32), (32, 64), (32, 32), (32, 16), (16, 32), (16, 16)]
_NL = len(_SIZES)
_PH = 64
_PW = 64
_PP = _PH * _PW
_NEG = -1e9
_BIG = 2 ** 30


def _head_nms_body(x_ref, w_ref, o_ref):
    l = pl.program_id(0)
    x = x_ref[0]
    w = w_ref[...]
    logits = jnp.dot(x, w, preferred_element_type=jnp.float32)

    heat = jnp.clip(jax.nn.sigmoid(logits[:, :2 * _NCLS]), 1e-4, 1.0 - 1e-4)

    h_l = sum((l == i) * h for i, (h, _) in enumerate(_SIZES))
    w_l = sum((l == i) * w_ for i, (_, w_) in enumerate(_SIZES))

    r = lax.broadcasted_iota(jnp.int32, (_PP, 1), 0)
    yy = r // _PW
    xx = r % _PW
    valid = (yy < h_l) & (xx < w_l)

    hv = jnp.where(valid, heat, _NEG)

    neg_row = jnp.full((1, 2 * _NCLS), _NEG, jnp.float32)
    left = jnp.where(xx == 0, _NEG,
                     jnp.concatenate([neg_row, hv[:-1, :]], axis=0))
    right = jnp.where(xx == _PW - 1, _NEG,
                      jnp.concatenate([hv[1:, :], neg_row], axis=0))
    hx = jnp.maximum(hv, jnp.maximum(left, right))
    neg_blk = jnp.full((_PW, 2 * _NCLS), _NEG, jnp.float32)
    up = jnp.concatenate([neg_blk, hx[:-_PW, :]], axis=0)
    dn = jnp.concatenate([hx[_PW:, :], neg_blk], axis=0)
    hmax = jnp.maximum(hx, jnp.maximum(up, dn))

    keep = (hmax == hv).astype(jnp.float32)
    scores = jnp.where(valid, heat * keep, 0.0)

    o_ref[0] = jnp.concatenate([scores, logits[:, 2 * _NCLS:]], axis=1)


def _heads(feats, w_all):
    return pl.pallas_call(
        _head_nms_body,
        grid=(_NL,),
        in_specs=[
            pl.BlockSpec((1, _PP, _CF), lambda l: (l, 0, 0)),
            pl.BlockSpec((_CF, _CF), lambda l: (0, 0)),
        ],
        out_specs=pl.BlockSpec((1, _PP, _CF), lambda l: (l, 0, 0)),
        out_shape=jax.ShapeDtypeStruct((_NL, _PP, _CF), jnp.float32),
    )(feats, w_all)


def _ext_step(k, carry, xs_ref, nb, forms):
    sub8 = lax.broadcasted_iota(jnp.int32, (8, 128), 0)
    lane8 = lax.broadcasted_iota(jnp.int32, (8, 128), 1)
    flat8 = sub8 * 128 + lane8
    biota = (lax.broadcasted_iota(jnp.int32, (nb, 128), 0) * 128 +
             lax.broadcasted_iota(jnp.int32, (nb, 128), 1))
    rowb = lax.broadcasted_iota(jnp.int32, (nb, 1), 0)

    m_sum, vacc, iacc = carry

    m = jnp.max(m_sum)
    ii = jnp.min(jnp.where(m_sum == m, biota, _BIG))
    r = ii // 128
    lane = ii % 128
    blk = xs_ref[pl.ds(r, 1)][0]
    hit = (blk == m) & (lane8 == lane)
    s = jnp.min(jnp.where(hit, sub8, _BIG))
    gidx = r * 1024 + s * 128 + lane
    sel = (sub8 == s) & (lane8 == lane)
    blk2 = jnp.where(sel, _NEG, blk)
    xs_ref[pl.ds(r, 1)] = blk2[None]
    newrow = jnp.max(blk2, axis=0, keepdims=True)
    m_sum = jnp.where(rowb == r, newrow, m_sum)
    vacc = jnp.where(flat8 == k, m, vacc)
    iacc = jnp.where(flat8 == k, gidx, iacc)
    return (m_sum, vacc, iacc)


def _init_summary(xs_ref, nb):
    xs = xs_ref[...]
    m_sum = xs[:, 0, :]
    for s in range(1, 8):
        m_sum = jnp.maximum(m_sum, xs[:, s, :])
    return m_sum


def _run_extraction(xs_ref, kk):
    nb = xs_ref.shape[0]
    m_sum = _init_summary(xs_ref, nb)
    init = (m_sum,
            jnp.full((8, 128), -1.0, jnp.float32),
            jnp.zeros((8, 128), jnp.int32))
    step = functools.partial(_ext_step, xs_ref=xs_ref, nb=nb, forms=False)

    def step4(t, c):
        for q in range(4):
            c = step(4 * t + q, c)
        return c

    _, vacc, iacc = lax.fori_loop(0, kk // 4, step4, init)
    return vacc, iacc


def _unit_step(k, carry, xs_ref, u, nb):
    sub8 = lax.broadcasted_iota(jnp.int32, (8, 128), 0)
    lane8 = lax.broadcasted_iota(jnp.int32, (8, 128), 1)
    flat8 = sub8 * 128 + lane8
    biota = (lax.broadcasted_iota(jnp.int32, (nb, 128), 0) * 128 +
             lax.broadcasted_iota(jnp.int32, (nb, 128), 1))
    rowb = lax.broadcasted_iota(jnp.int32, (nb, 1), 0)

    m_sum, vacc, iacc = carry
    m = jnp.max(m_sum)
    ii = jnp.min(jnp.where(m_sum == m, biota, _BIG))
    r = ii // 128
    lane = ii % 128
    blk = xs_ref[u, pl.ds(r, 1)][0]
    hit = (blk == m) & (lane8 == lane)
    s = jnp.min(jnp.where(hit, sub8, _BIG))
    gidx = r * 1024 + s * 128 + lane
    sel = (sub8 == s) & (lane8 == lane)
    blk2 = jnp.where(sel, _NEG, blk)
    xs_ref[u, pl.ds(r, 1)] = blk2[None]
    newrow = jnp.max(blk2, axis=0, keepdims=True)
    m_sum = jnp.where(rowb == r, newrow, m_sum)
    vacc = jnp.where(flat8 == k, m, vacc)
    iacc = jnp.where(flat8 == k, gidx, iacc)
    return (m_sum, vacc, iacc)


def _topk_units_body(x_ref, vals_ref, idx_ref, vcol_ref, icol_ref, xs_ref):
    nu, nb = x_ref.shape[0], x_ref.shape[1]
    xs_ref[...] = x_ref[...]
    msums, vaccs, iaccs = [], [], []
    for u in range(nu):
        xu = xs_ref[u]
        m = xu[:, 0, :]
        for s in range(1, 8):
            m = jnp.maximum(m, xu[:, s, :])
        msums.append(m)
        vaccs.append(jnp.full((8, 128), -1.0, jnp.float32))
        iaccs.append(jnp.zeros((8, 128), jnp.int32))

    def step(k, carry):
        ms, vs, is_ = carry
        out = [_unit_step(k, (ms[u], vs[u], is_[u]), xs_ref, u, nb)
               for u in range(nu)]
        return (tuple(o[0] for o in out), tuple(o[1] for o in out),
                tuple(o[2] for o in out))

    def step2(t, c):
        return step(2 * t + 1, step(2 * t, c))

    _, vaccs, iaccs = lax.fori_loop(
        0, _K // 2, step2, (tuple(msums), tuple(vaccs), tuple(iaccs)))

    io0 = lax.broadcasted_iota(jnp.int32, (128, 128), 0)
    io1 = lax.broadcasted_iota(jnp.int32, (128, 128), 1)
    eye = (io0 == io1).astype(jnp.float32)
    for u in range(nu):
        vals_ref[u] = vaccs[u]
        idx_ref[u] = iaccs[u]
        vt = lax.dot_general(eye, vaccs[u][0:1, :], (((1,), (1,)), ((), ())),
                             preferred_element_type=jnp.float32)
        it = lax.dot_general(eye, iaccs[u][0:1, :].astype(jnp.float32),
                             (((1,), (1,)), ((), ())),
                             preferred_element_type=jnp.float32)
        vcol_ref[u] = jnp.broadcast_to(vt, (128, 128))
        icol_ref[u] = jnp.broadcast_to(it.astype(jnp.int32), (128, 128))


def _final_body(keys_ref, ttl_ref, tbr_ref, out_ref, xs_ref):
    xs_ref[...] = keys_ref[...].reshape(_NL * 16, 8, 128)
    vacc, iacc = _run_extraction(xs_ref, 1000)

    io0 = lax.broadcasted_iota(jnp.int32, (128, 128), 0)
    io1 = lax.broadcasted_iota(jnp.int32, (128, 128), 1)
    eye = (io0 == io1).astype(jnp.float32)
    dn = (((1,), (1,)), ((), ()))
    f_col = jnp.concatenate(
        [lax.dot_general(eye, iacc[s:s + 1, :].astype(jnp.float32), dn,
                         preferred_element_type=jnp.float32)
         for s in range(8)], axis=0)
    v_col = jnp.concatenate(
        [lax.dot_general(eye, vacc[s:s + 1, :], dn,
                         preferred_element_type=jnp.float32)
         for s in range(8)], axis=0)

    f = f_col.astype(jnp.int32)
    l = f // 16384
    rem = f - l * 16384
    i = rem // 128
    j = rem - i * 128

    lane_row = lax.broadcasted_iota(jnp.int32, (1, 128), 1)
    acc = jnp.zeros((1024, 8), jnp.float32)
    for lidx in range(_NL):
        oh_tl = ((l == lidx) & (i == lane_row)).astype(jnp.float32)
        oh_br = ((l == lidx) & (j == lane_row)).astype(jnp.float32)
        a = jnp.dot(oh_tl, ttl_ref[lidx],
                    preferred_element_type=jnp.float32)
        b = lax.dot_general(oh_br, tbr_ref[lidx], dn,
                            preferred_element_type=jnp.float32)
        z4 = jnp.zeros((1024, 4), jnp.float32)
        acc = acc + jnp.concatenate([a[:, 0:2], b[:, 2:4], z4], axis=1)

    sc = jnp.where(v_col < 0.0, -1.0, v_col)
    z3 = jnp.zeros((1024, 3), jnp.float32)
    out_ref[...] = jnp.concatenate([acc[:, 0:4], sc, z3], axis=1)


def _topk_units(x):
    u, nb = x.shape[0], x.shape[1]
    return pl.pallas_call(
        _topk_units_body,
        out_shape=[
            jax.ShapeDtypeStruct((u, 8, 128), jnp.float32),
            jax.ShapeDtypeStruct((u, 8, 128), jnp.int32),
            jax.ShapeDtypeStruct((u, 128, 128), jnp.float32),
            jax.ShapeDtypeStruct((u, 128, 128), jnp.int32),
        ],
        scratch_shapes=[pltpu.VMEM((u, nb, 8, 128), jnp.float32)],
    )(x)


def _final(keys, ttl, tbr):
    return pl.pallas_call(
        _final_body,
        out_shape=jax.ShapeDtypeStruct((1024, 8), jnp.float32),
        scratch_shapes=[pltpu.VMEM((_NL * 16, 8, 128), jnp.float32)],
    )(keys, ttl, tbr)


def _div80(i):
    return ((i.astype(jnp.float32) + 0.5) * (1.0 / 80.0)).astype(jnp.int32)


def _pairs_body(vcol_ref, icol_ref, vrow_ref, irow_ref, proc_ref,
                keys_ref, ttl_ref, tbr_ref):
    l = pl.program_id(0)
    tl_s = vcol_ref[0][:, 0:1]
    tl_i = icol_ref[0][:, 0:1]
    br_s = vrow_ref[0][0:1, :]
    br_i = irow_ref[0][0:1, :]

    tl_p = _div80(tl_i)
    tl_c = tl_i - tl_p * 80
    br_p = _div80(br_i)
    br_c = br_i - br_p * 80
    tl_y = (tl_p // 64).astype(jnp.float32)
    tl_x = (tl_p % 64).astype(jnp.float32)
    br_y = (br_p // 64).astype(jnp.float32)
    br_x = (br_p % 64).astype(jnp.float32)

    regr = proc_ref[0][:, 2 * _NCLS:2 * _NCLS + 4]
    p_row = lax.broadcasted_iota(jnp.int32, (1, _PP), 1)
    p_col = lax.broadcasted_iota(jnp.int32, (_PP, 1), 0)
    oh_tl = (tl_p == p_row).astype(jnp.float32)
    rt = jnp.dot(oh_tl, regr, preferred_element_type=jnp.float32)
    oh_brT = (p_col == br_p).astype(jnp.float32)
    rb = lax.dot_general(regr, oh_brT, (((0,), (0,)), ((), ())),
                         preferred_element_type=jnp.float32)

    tlx = tl_x + rt[:, 0:1]
    tly = tl_y + rt[:, 1:2]
    brx = br_x + rb[2:3, :]
    bry = br_y + rb[3:4, :]

    i_io = lax.broadcasted_iota(jnp.int32, (128, 1), 0)
    j_io = lax.broadcasted_iota(jnp.int32, (1, 128), 1)
    in_k = (i_io < _K) & (j_io < _K)
    score = (tl_s + br_s) * 0.5
    bad = (tl_c != br_c) | (brx < tlx) | (bry < tly)
    semidx = (l * 10000 + i_io * 100 + j_io).astype(jnp.float32)
    inv_key = -1.0 - semidx * (2.0 ** -14)
    keys = jnp.where(in_k, jnp.where(bad, inv_key, score), _NEG)
    keys_ref[0] = keys

    z = jnp.zeros((128, 1), jnp.float32)
    ttl_ref[0] = jnp.concatenate([tlx, tly] + [z] * 6, axis=1)
    zr = jnp.zeros((1, 128), jnp.float32)
    tbr_ref[0] = jnp.concatenate([zr, zr, brx, bry] + [zr] * 4, axis=0)


def _pairs(vcol, icol, uvals, uidx, proc):
    return pl.pallas_call(
        _pairs_body,
        grid=(_NL,),
        in_specs=[
            pl.BlockSpec((1, 128, 128), lambda l: (l, 0, 0)),
            pl.BlockSpec((1, 128, 128), lambda l: (l, 0, 0)),
            pl.BlockSpec((1, 8, 128), lambda l: (l + _NL, 0, 0)),
            pl.BlockSpec((1, 8, 128), lambda l: (l + _NL, 0, 0)),
            pl.BlockSpec((1, _PP, _CF), lambda l: (l, 0, 0)),
        ],
        out_specs=[
            pl.BlockSpec((1, 128, 128), lambda l: (l, 0, 0)),
            pl.BlockSpec((1, 128, 8), lambda l: (l, 0, 0)),
            pl.BlockSpec((1, 8, 128), lambda l: (l, 0, 0)),
        ],
        out_shape=[
            jax.ShapeDtypeStruct((_NL, 128, 128), jnp.float32),
            jax.ShapeDtypeStruct((_NL, 128, 8), jnp.float32),
            jax.ShapeDtypeStruct((_NL, 8, 128), jnp.float32),
        ],
    )(vcol, icol, uvals, uidx, proc)


@jax.jit
def _run(feats, w_all):
    proc = _heads(feats, w_all)
    tl = proc[:, :, :_NCLS].reshape(_NL, 320, 8, 128)
    br = proc[:, :, _NCLS:2 * _NCLS].reshape(_NL, 320, 8, 128)
    units = jnp.concatenate([tl, br], axis=0)
    uvals, uidx, uvcol, uicol = _topk_units(units)
    keys, ttl, tbr = _pairs(uvcol, uicol, uvals, uidx, proc)
    out = _final(keys, ttl, tbr)
    det_scores = out[:1000, 4][None]
    det_boxes = out[:1000, :4][None]
    return det_scores, det_boxes


def kernel(feat_0, feat_1, feat_2, feat_3, feat_4, feat_5, feat_6,
           W_tl_heat, W_br_heat, W_tl_regr, W_br_regr):
    feats_in = [feat_0, feat_1, feat_2, feat_3, feat_4, feat_5, feat_6]
    padded = []
    for f, (h, w) in zip(feats_in, _SIZES):
        fm = jnp.transpose(f[0], (1, 2, 0))
        fm = jnp.pad(fm, ((0, _PH - h), (0, _PW - w), (0, 0)))
        padded.append(fm.reshape(_PP, _CF))
    feats = jnp.stack(padded, axis=0)
    w_all = jnp.concatenate(
        [W_tl_heat, W_br_heat, W_tl_regr, W_br_regr,
         jnp.zeros((_CF, _CF - 2 * _NCLS - 4), jnp.float32)], axis=1)
    return _run(feats, w_all)
